# baseline (speedup 1.0000x reference)
import functools

import jax
import jax.numpy as jnp
from jax import lax
from jax.experimental import pallas as pl
from jax.experimental.pallas import tpu as pltpu
from jax.experimental.pallas import tpu_sc as plsc

_LANES = 16
_SUB_WORDS = 16384


def _build_sc_add(batch, total_words):
    info = plsc.get_sparse_core_info()
    nc, ns = info.num_cores, info.num_subcores
    nw = nc * ns
    words_per_worker = total_words // nw
    nsub = words_per_worker // _SUB_WORDS
    mesh = plsc.VectorSubcoreMesh(core_axis_name="c", subcore_axis_name="s")

    @functools.partial(
        pl.kernel,
        mesh=mesh,
        out_type=jax.ShapeDtypeStruct((batch, total_words), jnp.float32),
        scratch_types=[
            pltpu.VMEM((_SUB_WORDS,), jnp.float32),
            pltpu.VMEM((_SUB_WORDS,), jnp.float32),
        ],
    )
    def sc_add(x_hbm, pos_hbm, out_hbm, pos_v, x_v):
        wid = lax.axis_index("s") * nc + lax.axis_index("c")
        for sub in range(nsub):
            off = wid * words_per_worker + sub * _SUB_WORDS
            pltpu.sync_copy(pos_hbm.at[pl.ds(off, _SUB_WORDS)], pos_v)
            for b in range(batch):
                pltpu.sync_copy(x_hbm.at[b, pl.ds(off, _SUB_WORDS)], x_v)

                def body(i, carry):
                    base = i * (_LANES * 8)
                    for j in range(8):
                        o = base + j * _LANES
                        x_v[pl.ds(o, _LANES)] = (
                            x_v[pl.ds(o, _LANES)] + pos_v[pl.ds(o, _LANES)]
                        )
                    return carry

                lax.fori_loop(0, _SUB_WORDS // (_LANES * 8), body, 0)
                pltpu.sync_copy(x_v, out_hbm.at[b, pl.ds(off, _SUB_WORDS)])

    return sc_add


@jax.jit
def kernel(x, pos_table):
    b, s, d = x.shape
    total = s * d
    xf = x.reshape(b, total)
    pf = pos_table.reshape(total)
    out = _build_sc_add(b, total)(xf, pf)
    return out.reshape(b, s, d)


# speedup vs baseline: 1.5756x; 1.5756x over previous
import functools

import jax
import jax.numpy as jnp
from jax import lax
from jax.experimental import pallas as pl
from jax.experimental.pallas import tpu as pltpu
from jax.experimental.pallas import tpu_sc as plsc

_LANES = 16
_SUB_ROWS = 16


def _build_sc_add(batch, seq, dim):
    info = plsc.get_sparse_core_info()
    nc, ns = info.num_cores, info.num_subcores
    nw = nc * ns
    rows_per_worker = seq // nw
    nsub = rows_per_worker // _SUB_ROWS
    slices_per_row = dim // _LANES
    mesh = plsc.VectorSubcoreMesh(core_axis_name="c", subcore_axis_name="s")

    @functools.partial(
        pl.kernel,
        mesh=mesh,
        out_type=jax.ShapeDtypeStruct((batch, seq, dim), jnp.float32),
        scratch_types=[
            pltpu.VMEM((_SUB_ROWS, dim), jnp.float32),
            pltpu.VMEM((_SUB_ROWS, dim), jnp.float32),
        ],
    )
    def sc_add(x_hbm, pos_hbm, out_hbm, pos_v, x_v):
        wid = lax.axis_index("s") * nc + lax.axis_index("c")
        for sub in range(nsub):
            r0 = wid * rows_per_worker + sub * _SUB_ROWS
            pltpu.sync_copy(pos_hbm.at[pl.ds(r0, _SUB_ROWS), :], pos_v)
            for b in range(batch):
                pltpu.sync_copy(x_hbm.at[b, pl.ds(r0, _SUB_ROWS), :], x_v)

                def body(r, carry):
                    for j in range(slices_per_row):
                        o = j * _LANES
                        x_v[r, pl.ds(o, _LANES)] = (
                            x_v[r, pl.ds(o, _LANES)] + pos_v[r, pl.ds(o, _LANES)]
                        )
                    return carry

                lax.fori_loop(0, _SUB_ROWS, body, 0)
                pltpu.sync_copy(x_v, out_hbm.at[b, pl.ds(r0, _SUB_ROWS), :])

    return sc_add


@jax.jit
def kernel(x, pos_table):
    b, s, d = x.shape
    return _build_sc_add(b, s, d)(x, pos_table)
